# Initial kernel scaffold; baseline (speedup 1.0000x reference)
import functools

import jax
import jax.numpy as jnp
import numpy as np
from jax import lax
from jax.experimental import pallas as pl
from jax.experimental.pallas import tpu as pltpu
from jax.experimental.pallas import tpu_sc as plsc

N_NODES = 10000
N_EDGES = 160000
D_IN = 256
D_HID = 256
N_CLASSES = 40
N_EXPERTS = 8

NUM_SC_CORES = 2
NUM_SUBCORES = 16
EDGES_PER_TILE = N_EDGES // NUM_SUBCORES
EDGE_BATCH = 80
BATCHES_PER_TILE = EDGES_PER_TILE // EDGE_BATCH
ROWS_PER_TILE = N_NODES // NUM_SUBCORES

EC = N_EXPERTS * N_CLASSES

NODE_BLOCK = 1000
NUM_NODE_BLOCKS = N_NODES // NODE_BLOCK


def _make_segsum(dh: int):
    mesh = plsc.VectorSubcoreMesh(
        core_axis_name="c", subcore_axis_name="s",
        num_cores=NUM_SC_CORES, num_subcores=NUM_SUBCORES)

    @functools.partial(
        pl.kernel,
        out_type=(
            jax.ShapeDtypeStruct((N_NODES, dh), jnp.float32),
            jax.ShapeDtypeStruct((N_NODES, dh), jnp.float32),
        ),
        mesh=mesh,
        scratch_types=[
            pltpu.VMEM((BATCHES_PER_TILE, EDGE_BATCH), jnp.int32),
            pltpu.VMEM((BATCHES_PER_TILE, EDGE_BATCH), jnp.int32),
            pltpu.VMEM((EDGE_BATCH, dh), jnp.float32),
            pltpu.VMEM_SHARED((N_NODES, dh), jnp.float32),
            pltpu.SemaphoreType.DMA,
        ],
    )
    def segsum(h0, h1, src_hbm, dst_hbm, zer_hbm, out0, out1,
               src_v, dst_v, rows_v, acc_sh, sem):
        c = lax.axis_index("c")
        s = lax.axis_index("s")

        pltpu.sync_copy(src_hbm.at[s], src_v)
        pltpu.sync_copy(dst_hbm.at[s], dst_v)
        row0 = s * ROWS_PER_TILE
        pltpu.sync_copy(zer_hbm.at[pl.ds(row0, ROWS_PER_TILE)],
                        acc_sh.at[pl.ds(row0, ROWS_PER_TILE)])
        plsc.subcore_barrier()

        def body(j, carry):
            @pl.when(c == 0)
            def _():
                pltpu.async_copy(h0.at[src_v.at[j]], rows_v, sem).wait()

            @pl.when(c == 1)
            def _():
                pltpu.async_copy(h1.at[src_v.at[j]], rows_v, sem).wait()

            pltpu.sync_copy(rows_v, acc_sh.at[dst_v.at[j]], add=True)
            return carry

        lax.fori_loop(0, BATCHES_PER_TILE, body, 0)
        plsc.subcore_barrier()

        @pl.when(c == 0)
        def _():
            pltpu.sync_copy(acc_sh.at[pl.ds(row0, ROWS_PER_TILE)],
                            out0.at[pl.ds(row0, ROWS_PER_TILE)])

        @pl.when(c == 1)
        def _():
            pltpu.sync_copy(acc_sh.at[pl.ds(row0, ROWS_PER_TILE)],
                            out1.at[pl.ds(row0, ROWS_PER_TILE)])

    return segsum


def _tc_experts_body(x_ref, agg0_ref, agg1_ref, noise_ref, wg_ref, wn_ref,
                     thr_ref, w1_ref, w2_ref, u_ref, gates_ref):
    x = x_ref[...]
    z = x + jnp.concatenate([agg0_ref[...], agg1_ref[...]], axis=1)

    clean = jnp.dot(x, wg_ref[...], preferred_element_type=jnp.float32)
    t = jnp.dot(x, wn_ref[...], preferred_element_type=jnp.float32)
    std = jnp.log1p(jnp.exp(-jnp.abs(t))) + jnp.maximum(t, 0.0) + 1e-2
    noisy = clean + noise_ref[...] * std
    scores = noisy - thr_ref[...]
    open_mask = (scores > 0.0).astype(jnp.float32)
    m = jnp.max(noisy, axis=1, keepdims=True)
    ex = jnp.exp(noisy - m)
    sm = ex / jnp.sum(ex, axis=1, keepdims=True)
    raw = sm * open_mask
    gates_ref[...] = raw / (jnp.sum(raw, axis=1, keepdims=True) + 1e-9)

    h = jnp.maximum(jnp.dot(z, w1_ref[...], preferred_element_type=jnp.float32),
                    0.0)
    u_ref[...] = jnp.dot(h, w2_ref[...], preferred_element_type=jnp.float32)


def _tc_experts(x, agg0, agg1, noise, w_gate, w_noise, thr, w1cat, w2bd):
    blk = lambda shape: pl.BlockSpec(shape, lambda i: (0, 0))
    return pl.pallas_call(
        _tc_experts_body,
        grid=(NUM_NODE_BLOCKS,),
        in_specs=[
            pl.BlockSpec((NODE_BLOCK, D_IN), lambda i: (i, 0)),
            pl.BlockSpec((NODE_BLOCK, D_IN // 2), lambda i: (i, 0)),
            pl.BlockSpec((NODE_BLOCK, D_IN // 2), lambda i: (i, 0)),
            pl.BlockSpec((NODE_BLOCK, N_EXPERTS), lambda i: (i, 0)),
            blk((D_IN, N_EXPERTS)),
            blk((D_IN, N_EXPERTS)),
            blk((1, N_EXPERTS)),
            blk((D_IN, N_EXPERTS * D_HID)),
            blk((N_EXPERTS * D_HID, EC)),
        ],
        out_specs=[
            pl.BlockSpec((NODE_BLOCK, EC), lambda i: (i, 0)),
            pl.BlockSpec((NODE_BLOCK, N_EXPERTS), lambda i: (i, 0)),
        ],
        out_shape=[
            jax.ShapeDtypeStruct((N_NODES, EC), jnp.float32),
            jax.ShapeDtypeStruct((N_NODES, N_EXPERTS), jnp.float32),
        ],
    )(x, agg0, agg1, noise, w_gate, w_noise, thr, w1cat, w2bd)


_REP_FULL = np.kron(np.eye(N_EXPERTS), np.ones((1, N_CLASSES))).astype(np.float32)
_SEL_FULL = np.kron(np.ones((N_EXPERTS, 1)), np.eye(N_CLASSES)).astype(np.float32)


def _tc_combine_body(u_ref, a0_ref, a1_ref, gates_ref, rep_ref, sel_ref,
                     repl_ref, sell_ref, reph_ref, selh_ref, y_ref):
    g = gates_ref[...]
    gf = jnp.dot(g, rep_ref[...], preferred_element_type=jnp.float32)
    gl = jnp.dot(g, repl_ref[...], preferred_element_type=jnp.float32)
    gh = jnp.dot(g, reph_ref[...], preferred_element_type=jnp.float32)
    y = jnp.dot(u_ref[...] * gf, sel_ref[...],
                preferred_element_type=jnp.float32)
    y += jnp.dot(a0_ref[...] * gl, sell_ref[...],
                 preferred_element_type=jnp.float32)
    y += jnp.dot(a1_ref[...] * gh, selh_ref[...],
                 preferred_element_type=jnp.float32)
    y_ref[...] = y


def _tc_combine(u, a0, a1, gates):
    half = EC // 2
    blk = lambda shape: pl.BlockSpec(shape, lambda i: (0, 0))
    consts = (
        jnp.asarray(_REP_FULL), jnp.asarray(_SEL_FULL),
        jnp.asarray(_REP_FULL[:, :half]), jnp.asarray(_SEL_FULL[:half]),
        jnp.asarray(_REP_FULL[:, half:]), jnp.asarray(_SEL_FULL[half:]),
    )
    return pl.pallas_call(
        _tc_combine_body,
        grid=(NUM_NODE_BLOCKS,),
        in_specs=[
            pl.BlockSpec((NODE_BLOCK, EC), lambda i: (i, 0)),
            pl.BlockSpec((NODE_BLOCK, half), lambda i: (i, 0)),
            pl.BlockSpec((NODE_BLOCK, half), lambda i: (i, 0)),
            pl.BlockSpec((NODE_BLOCK, N_EXPERTS), lambda i: (i, 0)),
            blk((N_EXPERTS, EC)), blk((EC, N_CLASSES)),
            blk((N_EXPERTS, half)), blk((half, N_CLASSES)),
            blk((N_EXPERTS, half)), blk((half, N_CLASSES)),
        ],
        out_specs=pl.BlockSpec((NODE_BLOCK, N_CLASSES), lambda i: (i, 0)),
        out_shape=jax.ShapeDtypeStruct((N_NODES, N_CLASSES), jnp.float32),
    )(u, a0, a1, gates, *consts)


def kernel(x, edge_index, noise, w_gate, w_noise, gate_threshold, W1, W2):
    src = edge_index[0].astype(jnp.int32).reshape(
        NUM_SUBCORES, BATCHES_PER_TILE, EDGE_BATCH)
    dst = edge_index[1].astype(jnp.int32).reshape(
        NUM_SUBCORES, BATCHES_PER_TILE, EDGE_BATCH)

    xh = D_IN // 2
    x0 = x[:, :xh]
    x1 = x[:, xh:]
    zer_x = jnp.zeros((N_NODES, xh), jnp.float32)
    agg0, agg1 = _make_segsum(xh)(x0, x1, src, dst, zer_x)

    w1cat = jnp.transpose(W1, (1, 0, 2)).reshape(D_IN, N_EXPERTS * D_HID)
    w2bd = jax.scipy.linalg.block_diag(*[W2[e] for e in range(N_EXPERTS)])
    thr = gate_threshold.reshape(1, N_EXPERTS)

    u, gates = _tc_experts(x, agg0, agg1, noise, w_gate, w_noise, thr,
                           w1cat, w2bd)

    uh = EC // 2
    u0 = u[:, :uh]
    u1 = u[:, uh:]
    zer_u = jnp.zeros((N_NODES, uh), jnp.float32)
    a0, a1 = _make_segsum(uh)(u0, u1, src, dst, zer_u)

    return _tc_combine(u, a0, a1, gates)


# speedup vs baseline: 7.2150x; 7.2150x over previous
import functools

import jax
import jax.numpy as jnp
import numpy as np
from jax import lax
from jax.experimental import pallas as pl
from jax.experimental.pallas import tpu as pltpu
from jax.experimental.pallas import tpu_sc as plsc

N_NODES = 10000
N_EDGES = 160000
D_IN = 256
D_HID = 256
N_CLASSES = 40
N_EXPERTS = 8

NUM_SC_CORES = 2
NUM_SUBCORES = 16
EDGES_PER_TILE = N_EDGES // NUM_SUBCORES
EDGE_BATCH = 80
BATCHES_PER_TILE = EDGES_PER_TILE // EDGE_BATCH
ROWS_PER_TILE = 624
ROWS_LAST_TILE = N_NODES - ROWS_PER_TILE * (NUM_SUBCORES - 1)

EC = N_EXPERTS * N_CLASSES

NODE_BLOCK = 1000
NUM_NODE_BLOCKS = N_NODES // NODE_BLOCK


@functools.cache
def _mesh():
    return plsc.VectorSubcoreMesh(
        core_axis_name="c", subcore_axis_name="s",
        num_cores=NUM_SC_CORES, num_subcores=NUM_SUBCORES)


DH = 128


def _zero_my_rows(zer_hbm, acc_sh, s):
    row0 = s * ROWS_PER_TILE

    @pl.when(s < NUM_SUBCORES - 1)
    def _():
        pltpu.sync_copy(zer_hbm.at[pl.ds(row0, ROWS_PER_TILE)],
                        acc_sh.at[pl.ds(row0, ROWS_PER_TILE)])

    @pl.when(s == NUM_SUBCORES - 1)
    def _():
        pltpu.sync_copy(zer_hbm.at[pl.ds(row0, ROWS_LAST_TILE)],
                        acc_sh.at[pl.ds(row0, ROWS_LAST_TILE)])


def _writeback_my_rows(acc_sh, out, s):
    row0 = s * ROWS_PER_TILE

    @pl.when(s < NUM_SUBCORES - 1)
    def _():
        pltpu.sync_copy(acc_sh.at[pl.ds(row0, ROWS_PER_TILE)],
                        out.at[pl.ds(row0, ROWS_PER_TILE)])

    @pl.when(s == NUM_SUBCORES - 1)
    def _():
        pltpu.sync_copy(acc_sh.at[pl.ds(row0, ROWS_LAST_TILE)],
                        out.at[pl.ds(row0, ROWS_LAST_TILE)])


W_BATCH = 40
W_BATCHES = N_EDGES // (NUM_SC_CORES * NUM_SUBCORES * W_BATCH)


@functools.cache
def _build_segsum():
    @functools.partial(
        pl.kernel,
        out_type=(
            jax.ShapeDtypeStruct((N_NODES, DH), jnp.float32),
            jax.ShapeDtypeStruct((N_NODES, DH), jnp.float32),
        ),
        mesh=_mesh(),
        scratch_types=[
            pltpu.VMEM((W_BATCHES, W_BATCH), jnp.int32),
            pltpu.VMEM((W_BATCHES, W_BATCH), jnp.int32),
            pltpu.VMEM((W_BATCH, DH), jnp.float32),
            pltpu.VMEM_SHARED((N_NODES, DH), jnp.float32),
            pltpu.SemaphoreType.DMA,
        ],
    )
    def segsum(h, src_hbm, dst_hbm, zer_hbm, outa, outb,
               src_v, dst_v, rows_v, acc_sh, sem):
        c = lax.axis_index("c")
        s = lax.axis_index("s")

        pltpu.sync_copy(src_hbm.at[c, s], src_v)
        pltpu.sync_copy(dst_hbm.at[c, s], dst_v)
        _zero_my_rows(zer_hbm, acc_sh, s)
        plsc.subcore_barrier()

        def body(j, carry):
            pltpu.async_copy(h.at[src_v.at[j]], rows_v, sem).wait()
            pltpu.sync_copy(rows_v, acc_sh.at[dst_v.at[j]], add=True)
            return carry

        lax.fori_loop(0, W_BATCHES, body, 0)
        plsc.subcore_barrier()

        @pl.when(c == 0)
        def _():
            _writeback_my_rows(acc_sh, outa, s)

        @pl.when(c == 1)
        def _():
            _writeback_my_rows(acc_sh, outb, s)

    return segsum


def _segsum(h, src, dst, zer):
    return _build_segsum()(h, src, dst, zer)


def _tc_experts_body(x_ref, a0a_ref, a0b_ref, a1a_ref, a1b_ref, noise_ref,
                     wg_ref, wn_ref, thr_ref, w1_ref, w2_ref, u_ref,
                     gates_ref):
    x = x_ref[...]
    z = x + jnp.concatenate([a0a_ref[...] + a0b_ref[...],
                             a1a_ref[...] + a1b_ref[...]], axis=1)

    clean = jnp.dot(x, wg_ref[...], preferred_element_type=jnp.float32)
    t = jnp.dot(x, wn_ref[...], preferred_element_type=jnp.float32)
    std = jnp.log1p(jnp.exp(-jnp.abs(t))) + jnp.maximum(t, 0.0) + 1e-2
    noisy = clean + noise_ref[...] * std
    scores = noisy - thr_ref[...]
    open_mask = (scores > 0.0).astype(jnp.float32)
    m = jnp.max(noisy, axis=1, keepdims=True)
    ex = jnp.exp(noisy - m)
    sm = ex / jnp.sum(ex, axis=1, keepdims=True)
    raw = sm * open_mask
    gates_ref[...] = raw / (jnp.sum(raw, axis=1, keepdims=True) + 1e-9)

    h = jnp.maximum(jnp.dot(z, w1_ref[...], preferred_element_type=jnp.float32),
                    0.0)
    u_ref[...] = jnp.dot(h, w2_ref[...], preferred_element_type=jnp.float32)


def _tc_experts(x, a0a, a0b, a1a, a1b, noise, w_gate, w_noise, thr,
                w1cat, w2bd):
    blk = lambda shape: pl.BlockSpec(shape, lambda i: (0, 0))
    return pl.pallas_call(
        _tc_experts_body,
        grid=(NUM_NODE_BLOCKS,),
        in_specs=[
            pl.BlockSpec((NODE_BLOCK, D_IN), lambda i: (i, 0)),
            pl.BlockSpec((NODE_BLOCK, DH), lambda i: (i, 0)),
            pl.BlockSpec((NODE_BLOCK, DH), lambda i: (i, 0)),
            pl.BlockSpec((NODE_BLOCK, DH), lambda i: (i, 0)),
            pl.BlockSpec((NODE_BLOCK, DH), lambda i: (i, 0)),
            pl.BlockSpec((NODE_BLOCK, N_EXPERTS), lambda i: (i, 0)),
            blk((D_IN, N_EXPERTS)),
            blk((D_IN, N_EXPERTS)),
            blk((1, N_EXPERTS)),
            blk((D_IN, N_EXPERTS * D_HID)),
            blk((N_EXPERTS * D_HID, EC)),
        ],
        out_specs=[
            pl.BlockSpec((NODE_BLOCK, EC), lambda i: (i, 0)),
            pl.BlockSpec((NODE_BLOCK, N_EXPERTS), lambda i: (i, 0)),
        ],
        out_shape=[
            jax.ShapeDtypeStruct((N_NODES, EC), jnp.float32),
            jax.ShapeDtypeStruct((N_NODES, N_EXPERTS), jnp.float32),
        ],
    )(x, a0a, a0b, a1a, a1b, noise, w_gate, w_noise, thr, w1cat, w2bd)


_REP_FULL = np.kron(np.eye(N_EXPERTS), np.ones((1, N_CLASSES))).astype(np.float32)
_SEL_FULL = np.kron(np.ones((N_EXPERTS, 1)), np.eye(N_CLASSES)).astype(np.float32)


_REP_PAD = np.concatenate(
    [_REP_FULL, np.zeros((N_EXPERTS, 3 * DH - EC), np.float32)], axis=1)
_SEL_PAD = np.concatenate(
    [_SEL_FULL, np.zeros((3 * DH - EC, N_CLASSES), np.float32)], axis=0)


def _tc_combine_body(u_ref, p0a_ref, p0b_ref, p1a_ref, p1b_ref, p2a_ref,
                     p2b_ref, gates_ref, rep_ref, sel_ref, repp_ref,
                     selp_ref, y_ref):
    g = gates_ref[...]
    a2 = jnp.concatenate(
        [p0a_ref[...] + p0b_ref[...], p1a_ref[...] + p1b_ref[...],
         p2a_ref[...] + p2b_ref[...]], axis=1)
    gf = jnp.dot(g, rep_ref[...], preferred_element_type=jnp.float32)
    gp = jnp.dot(g, repp_ref[...], preferred_element_type=jnp.float32)
    y = jnp.dot(u_ref[...] * gf, sel_ref[...],
                preferred_element_type=jnp.float32)
    y += jnp.dot(a2 * gp, selp_ref[...], preferred_element_type=jnp.float32)
    y_ref[...] = y


def _tc_combine(u, p0a, p0b, p1a, p1b, p2a, p2b, gates):
    blk = lambda shape: pl.BlockSpec(shape, lambda i: (0, 0))
    consts = (
        jnp.asarray(_REP_FULL), jnp.asarray(_SEL_FULL),
        jnp.asarray(_REP_PAD), jnp.asarray(_SEL_PAD),
    )
    return pl.pallas_call(
        _tc_combine_body,
        grid=(NUM_NODE_BLOCKS,),
        in_specs=[
            pl.BlockSpec((NODE_BLOCK, EC), lambda i: (i, 0)),
            pl.BlockSpec((NODE_BLOCK, DH), lambda i: (i, 0)),
            pl.BlockSpec((NODE_BLOCK, DH), lambda i: (i, 0)),
            pl.BlockSpec((NODE_BLOCK, DH), lambda i: (i, 0)),
            pl.BlockSpec((NODE_BLOCK, DH), lambda i: (i, 0)),
            pl.BlockSpec((NODE_BLOCK, DH), lambda i: (i, 0)),
            pl.BlockSpec((NODE_BLOCK, DH), lambda i: (i, 0)),
            pl.BlockSpec((NODE_BLOCK, N_EXPERTS), lambda i: (i, 0)),
            blk((N_EXPERTS, EC)), blk((EC, N_CLASSES)),
            blk((N_EXPERTS, 3 * DH)), blk((3 * DH, N_CLASSES)),
        ],
        out_specs=pl.BlockSpec((NODE_BLOCK, N_CLASSES), lambda i: (i, 0)),
        out_shape=jax.ShapeDtypeStruct((N_NODES, N_CLASSES), jnp.float32),
    )(u, p0a, p0b, p1a, p1b, p2a, p2b, gates, *consts)


def kernel(x, edge_index, noise, w_gate, w_noise, gate_threshold, W1, W2):
    src = edge_index[0].astype(jnp.int32).reshape(
        NUM_SC_CORES, NUM_SUBCORES, W_BATCHES, W_BATCH)
    dst = edge_index[1].astype(jnp.int32).reshape(
        NUM_SC_CORES, NUM_SUBCORES, W_BATCHES, W_BATCH)

    zer = jnp.zeros((N_NODES, DH), jnp.float32)
    x0 = x[:, :DH]
    x1 = x[:, DH:]
    x0a, x0b = _segsum(x0, src, dst, zer)
    x1a, x1b = _segsum(x1, src, dst, zer)

    w1cat = jnp.transpose(W1, (1, 0, 2)).reshape(D_IN, N_EXPERTS * D_HID)
    w2bd = jax.scipy.linalg.block_diag(*[W2[e] for e in range(N_EXPERTS)])
    thr = gate_threshold.reshape(1, N_EXPERTS)

    u, gates = _tc_experts(x, x0a, x0b, x1a, x1b, noise, w_gate, w_noise,
                           thr, w1cat, w2bd)

    u0 = u[:, :DH]
    u1 = u[:, DH:2 * DH]
    u2 = jnp.pad(u[:, 2 * DH:], ((0, 0), (0, 3 * DH - EC)))
    p0a, p0b = _segsum(u0, src, dst, zer)
    p1a, p1b = _segsum(u1, src, dst, zer)
    p2a, p2b = _segsum(u2, src, dst, zer)

    return _tc_combine(u, p0a, p0b, p1a, p1b, p2a, p2b, gates)


# speedup vs baseline: 15.2590x; 2.1149x over previous
import functools

import jax
import jax.numpy as jnp
import numpy as np
from jax import lax
from jax.experimental import pallas as pl
from jax.experimental.pallas import tpu as pltpu
from jax.experimental.pallas import tpu_sc as plsc

N_NODES = 10000
N_EDGES = 160000
D_IN = 256
D_HID = 256
N_CLASSES = 40
N_EXPERTS = 8

NUM_SC_CORES = 2
NUM_SUBCORES = 16
EDGES_PER_TILE = N_EDGES // NUM_SUBCORES
EDGE_BATCH = 80
BATCHES_PER_TILE = EDGES_PER_TILE // EDGE_BATCH
ROWS_PER_TILE = 624
ROWS_LAST_TILE = N_NODES - ROWS_PER_TILE * (NUM_SUBCORES - 1)

EC = N_EXPERTS * N_CLASSES

NODE_BLOCK = 1000
NUM_NODE_BLOCKS = N_NODES // NODE_BLOCK


@functools.cache
def _mesh():
    return plsc.VectorSubcoreMesh(
        core_axis_name="c", subcore_axis_name="s",
        num_cores=NUM_SC_CORES, num_subcores=NUM_SUBCORES)


DH = 128


def _zero_my_rows(zer_hbm, acc_sh, s):
    row0 = s * ROWS_PER_TILE

    @pl.when(s < NUM_SUBCORES - 1)
    def _():
        pltpu.sync_copy(zer_hbm.at[pl.ds(row0, ROWS_PER_TILE)],
                        acc_sh.at[pl.ds(row0, ROWS_PER_TILE)])

    @pl.when(s == NUM_SUBCORES - 1)
    def _():
        pltpu.sync_copy(zer_hbm.at[pl.ds(row0, ROWS_LAST_TILE)],
                        acc_sh.at[pl.ds(row0, ROWS_LAST_TILE)])


def _writeback_my_rows(acc_sh, out, s):
    row0 = s * ROWS_PER_TILE

    @pl.when(s < NUM_SUBCORES - 1)
    def _():
        pltpu.sync_copy(acc_sh.at[pl.ds(row0, ROWS_PER_TILE)],
                        out.at[pl.ds(row0, ROWS_PER_TILE)])

    @pl.when(s == NUM_SUBCORES - 1)
    def _():
        pltpu.sync_copy(acc_sh.at[pl.ds(row0, ROWS_LAST_TILE)],
                        out.at[pl.ds(row0, ROWS_LAST_TILE)])


EDGES_PER_WORKER = N_EDGES // (NUM_SC_CORES * NUM_SUBCORES)
MAIN_BATCH = 80
N_MAIN = 62
TAIL_BATCH = EDGES_PER_WORKER - N_MAIN * MAIN_BATCH


@functools.cache
def _build_segsum():
    @functools.partial(
        pl.kernel,
        out_type=(
            jax.ShapeDtypeStruct((N_NODES, DH), jnp.float32),
            jax.ShapeDtypeStruct((N_NODES, DH), jnp.float32),
        ),
        mesh=_mesh(),
        scratch_types=[
            pltpu.VMEM((N_MAIN, MAIN_BATCH), jnp.int32),
            pltpu.VMEM((N_MAIN, MAIN_BATCH), jnp.int32),
            pltpu.VMEM((1, TAIL_BATCH), jnp.int32),
            pltpu.VMEM((1, TAIL_BATCH), jnp.int32),
            pltpu.VMEM((MAIN_BATCH, DH), jnp.float32),
            pltpu.VMEM((MAIN_BATCH, DH), jnp.float32),
            pltpu.VMEM((MAIN_BATCH, DH), jnp.float32),
            pltpu.VMEM_SHARED((N_NODES, DH), jnp.float32),
            pltpu.SemaphoreType.DMA,
            pltpu.SemaphoreType.DMA,
            pltpu.SemaphoreType.DMA,
            pltpu.SemaphoreType.DMA,
        ],
    )
    def segsum(h, srcm_hbm, dstm_hbm, srct_hbm, dstt_hbm, zer_hbm,
               outa, outb,
               srcm_v, dstm_v, srct_v, dstt_v, r0, r1, r2, acc_sh,
               g0, g1, g2, ss):
        c = lax.axis_index("c")
        s = lax.axis_index("s")
        bufs = (r0, r1, r2)
        gsems = (g0, g1, g2)

        pltpu.sync_copy(srcm_hbm.at[c, s], srcm_v)
        pltpu.sync_copy(dstm_hbm.at[c, s], dstm_v)
        pltpu.sync_copy(srct_hbm.at[c, s], srct_v)
        pltpu.sync_copy(dstt_hbm.at[c, s], dstt_v)

        pltpu.async_copy(h.at[srcm_v.at[0]], r0, g0)
        pltpu.async_copy(h.at[srcm_v.at[1]], r1, g1)
        _zero_my_rows(zer_hbm, acc_sh, s)
        plsc.subcore_barrier()

        def step(j, cur, gcur, prv, tw0):
            pltpu.make_async_copy(h.at[srcm_v.at[j]], cur, gcur).wait()

            @pl.when(j >= 1)
            def _():
                pltpu.make_async_copy(
                    prv, acc_sh.at[dstm_v.at[j - 1]], ss).wait()

            @pl.when(j < N_MAIN - 2)
            def _():
                pltpu.async_copy(h.at[srcm_v.at[j + 2]], prv, tw0)

            pltpu.async_copy(cur, acc_sh.at[dstm_v.at[j]], ss, add=True)

        def body(j, carry):
            @pl.when(j % 3 == 0)
            def _():
                step(j, r0, g0, r2, g2)

            @pl.when(j % 3 == 1)
            def _():
                step(j, r1, g1, r0, g0)

            @pl.when(j % 3 == 2)
            def _():
                step(j, r2, g2, r1, g1)

            return carry

        lax.fori_loop(0, N_MAIN, body, 0)

        pltpu.make_async_copy(
            bufs[(N_MAIN - 1) % 3],
            acc_sh.at[dstm_v.at[N_MAIN - 1]], ss).wait()
        rtail = r0.at[pl.ds(0, TAIL_BATCH)]
        pltpu.async_copy(h.at[srct_v.at[0]], rtail, g0).wait()
        pltpu.sync_copy(rtail, acc_sh.at[dstt_v.at[0]], add=True)
        plsc.subcore_barrier()

        @pl.when(c == 0)
        def _():
            _writeback_my_rows(acc_sh, outa, s)

        @pl.when(c == 1)
        def _():
            _writeback_my_rows(acc_sh, outb, s)

    return segsum


def _segsum(h, srcm, dstm, srct, dstt, zer):
    return _build_segsum()(h, srcm, dstm, srct, dstt, zer)


def _tc_experts_body(x_ref, a0a_ref, a0b_ref, a1a_ref, a1b_ref, noise_ref,
                     wg_ref, wn_ref, thr_ref, w1_ref, w2_ref, u_ref,
                     gates_ref):
    x = x_ref[...]
    z = x + jnp.concatenate([a0a_ref[...] + a0b_ref[...],
                             a1a_ref[...] + a1b_ref[...]], axis=1)

    clean = jnp.dot(x, wg_ref[...], preferred_element_type=jnp.float32)
    t = jnp.dot(x, wn_ref[...], preferred_element_type=jnp.float32)
    std = jnp.log1p(jnp.exp(-jnp.abs(t))) + jnp.maximum(t, 0.0) + 1e-2
    noisy = clean + noise_ref[...] * std
    scores = noisy - thr_ref[...]
    open_mask = (scores > 0.0).astype(jnp.float32)
    m = jnp.max(noisy, axis=1, keepdims=True)
    ex = jnp.exp(noisy - m)
    sm = ex / jnp.sum(ex, axis=1, keepdims=True)
    raw = sm * open_mask
    gates_ref[...] = raw / (jnp.sum(raw, axis=1, keepdims=True) + 1e-9)

    h = jnp.maximum(jnp.dot(z, w1_ref[...], preferred_element_type=jnp.float32),
                    0.0)
    u_ref[...] = jnp.dot(h, w2_ref[...], preferred_element_type=jnp.float32)


def _tc_experts(x, a0a, a0b, a1a, a1b, noise, w_gate, w_noise, thr,
                w1cat, w2bd):
    blk = lambda shape: pl.BlockSpec(shape, lambda i: (0, 0))
    return pl.pallas_call(
        _tc_experts_body,
        grid=(NUM_NODE_BLOCKS,),
        in_specs=[
            pl.BlockSpec((NODE_BLOCK, D_IN), lambda i: (i, 0)),
            pl.BlockSpec((NODE_BLOCK, DH), lambda i: (i, 0)),
            pl.BlockSpec((NODE_BLOCK, DH), lambda i: (i, 0)),
            pl.BlockSpec((NODE_BLOCK, DH), lambda i: (i, 0)),
            pl.BlockSpec((NODE_BLOCK, DH), lambda i: (i, 0)),
            pl.BlockSpec((NODE_BLOCK, N_EXPERTS), lambda i: (i, 0)),
            blk((D_IN, N_EXPERTS)),
            blk((D_IN, N_EXPERTS)),
            blk((1, N_EXPERTS)),
            blk((D_IN, N_EXPERTS * D_HID)),
            blk((N_EXPERTS * D_HID, EC)),
        ],
        out_specs=[
            pl.BlockSpec((NODE_BLOCK, EC), lambda i: (i, 0)),
            pl.BlockSpec((NODE_BLOCK, N_EXPERTS), lambda i: (i, 0)),
        ],
        out_shape=[
            jax.ShapeDtypeStruct((N_NODES, EC), jnp.float32),
            jax.ShapeDtypeStruct((N_NODES, N_EXPERTS), jnp.float32),
        ],
    )(x, a0a, a0b, a1a, a1b, noise, w_gate, w_noise, thr, w1cat, w2bd)


_REP_FULL = np.kron(np.eye(N_EXPERTS), np.ones((1, N_CLASSES))).astype(np.float32)
_SEL_FULL = np.kron(np.ones((N_EXPERTS, 1)), np.eye(N_CLASSES)).astype(np.float32)


_REP_PAD = np.concatenate(
    [_REP_FULL, np.zeros((N_EXPERTS, 3 * DH - EC), np.float32)], axis=1)
_SEL_PAD = np.concatenate(
    [_SEL_FULL, np.zeros((3 * DH - EC, N_CLASSES), np.float32)], axis=0)


def _tc_combine_body(u_ref, p0a_ref, p0b_ref, p1a_ref, p1b_ref, p2a_ref,
                     p2b_ref, gates_ref, rep_ref, sel_ref, repp_ref,
                     selp_ref, y_ref):
    g = gates_ref[...]
    a2 = jnp.concatenate(
        [p0a_ref[...] + p0b_ref[...], p1a_ref[...] + p1b_ref[...],
         p2a_ref[...] + p2b_ref[...]], axis=1)
    gf = jnp.dot(g, rep_ref[...], preferred_element_type=jnp.float32)
    gp = jnp.dot(g, repp_ref[...], preferred_element_type=jnp.float32)
    y = jnp.dot(u_ref[...] * gf, sel_ref[...],
                preferred_element_type=jnp.float32)
    y += jnp.dot(a2 * gp, selp_ref[...], preferred_element_type=jnp.float32)
    y_ref[...] = y


def _tc_combine(u, p0a, p0b, p1a, p1b, p2a, p2b, gates):
    blk = lambda shape: pl.BlockSpec(shape, lambda i: (0, 0))
    consts = (
        jnp.asarray(_REP_FULL), jnp.asarray(_SEL_FULL),
        jnp.asarray(_REP_PAD), jnp.asarray(_SEL_PAD),
    )
    return pl.pallas_call(
        _tc_combine_body,
        grid=(NUM_NODE_BLOCKS,),
        in_specs=[
            pl.BlockSpec((NODE_BLOCK, EC), lambda i: (i, 0)),
            pl.BlockSpec((NODE_BLOCK, DH), lambda i: (i, 0)),
            pl.BlockSpec((NODE_BLOCK, DH), lambda i: (i, 0)),
            pl.BlockSpec((NODE_BLOCK, DH), lambda i: (i, 0)),
            pl.BlockSpec((NODE_BLOCK, DH), lambda i: (i, 0)),
            pl.BlockSpec((NODE_BLOCK, DH), lambda i: (i, 0)),
            pl.BlockSpec((NODE_BLOCK, DH), lambda i: (i, 0)),
            pl.BlockSpec((NODE_BLOCK, N_EXPERTS), lambda i: (i, 0)),
            blk((N_EXPERTS, EC)), blk((EC, N_CLASSES)),
            blk((N_EXPERTS, 3 * DH)), blk((3 * DH, N_CLASSES)),
        ],
        out_specs=pl.BlockSpec((NODE_BLOCK, N_CLASSES), lambda i: (i, 0)),
        out_shape=jax.ShapeDtypeStruct((N_NODES, N_CLASSES), jnp.float32),
    )(u, p0a, p0b, p1a, p1b, p2a, p2b, gates, *consts)


def kernel(x, edge_index, noise, w_gate, w_noise, gate_threshold, W1, W2):
    ncut = N_MAIN * MAIN_BATCH
    src = edge_index[0].astype(jnp.int32).reshape(
        NUM_SC_CORES, NUM_SUBCORES, EDGES_PER_WORKER)
    dst = edge_index[1].astype(jnp.int32).reshape(
        NUM_SC_CORES, NUM_SUBCORES, EDGES_PER_WORKER)
    srcm = src[:, :, :ncut].reshape(NUM_SC_CORES, NUM_SUBCORES,
                                    N_MAIN, MAIN_BATCH)
    dstm = dst[:, :, :ncut].reshape(NUM_SC_CORES, NUM_SUBCORES,
                                    N_MAIN, MAIN_BATCH)
    srct = src[:, :, ncut:].reshape(NUM_SC_CORES, NUM_SUBCORES,
                                    1, TAIL_BATCH)
    dstt = dst[:, :, ncut:].reshape(NUM_SC_CORES, NUM_SUBCORES,
                                    1, TAIL_BATCH)
    idx = (srcm, dstm, srct, dstt)

    zer = jnp.zeros((N_NODES, DH), jnp.float32)
    x0 = x[:, :DH]
    x1 = x[:, DH:]
    x0a, x0b = _segsum(x0, *idx, zer)
    x1a, x1b = _segsum(x1, *idx, zer)

    w1cat = jnp.transpose(W1, (1, 0, 2)).reshape(D_IN, N_EXPERTS * D_HID)
    w2bd = jax.scipy.linalg.block_diag(*[W2[e] for e in range(N_EXPERTS)])
    thr = gate_threshold.reshape(1, N_EXPERTS)

    u, gates = _tc_experts(x, x0a, x0b, x1a, x1b, noise, w_gate, w_noise,
                           thr, w1cat, w2bd)

    u0 = u[:, :DH]
    u1 = u[:, DH:2 * DH]
    u2 = jnp.pad(u[:, 2 * DH:], ((0, 0), (0, 3 * DH - EC)))
    p0a, p0b = _segsum(u0, *idx, zer)
    p1a, p1b = _segsum(u1, *idx, zer)
    p2a, p2b = _segsum(u2, *idx, zer)

    return _tc_combine(u, p0a, p0b, p1a, p1b, p2a, p2b, gates)


# speedup vs baseline: 15.5821x; 1.0212x over previous
import functools

import jax
import jax.numpy as jnp
import numpy as np
from jax import lax
from jax.experimental import pallas as pl
from jax.experimental.pallas import tpu as pltpu
from jax.experimental.pallas import tpu_sc as plsc

N_NODES = 10000
N_EDGES = 160000
D_IN = 256
D_HID = 256
N_CLASSES = 40
N_EXPERTS = 8

NUM_SC_CORES = 2
NUM_SUBCORES = 16
EDGES_PER_TILE = N_EDGES // NUM_SUBCORES
EDGE_BATCH = 80
BATCHES_PER_TILE = EDGES_PER_TILE // EDGE_BATCH
ROWS_PER_TILE = 624
ROWS_LAST_TILE = N_NODES - ROWS_PER_TILE * (NUM_SUBCORES - 1)

EC = N_EXPERTS * N_CLASSES

NODE_BLOCK = 1000
NUM_NODE_BLOCKS = N_NODES // NODE_BLOCK


@functools.cache
def _mesh():
    return plsc.VectorSubcoreMesh(
        core_axis_name="c", subcore_axis_name="s",
        num_cores=NUM_SC_CORES, num_subcores=NUM_SUBCORES)


DH = 128


def _zero_my_rows(zer_hbm, acc_sh, s):
    row0 = s * ROWS_PER_TILE

    @pl.when(s < NUM_SUBCORES - 1)
    def _():
        pltpu.sync_copy(zer_hbm.at[pl.ds(row0, ROWS_PER_TILE)],
                        acc_sh.at[pl.ds(row0, ROWS_PER_TILE)])

    @pl.when(s == NUM_SUBCORES - 1)
    def _():
        pltpu.sync_copy(zer_hbm.at[pl.ds(row0, ROWS_LAST_TILE)],
                        acc_sh.at[pl.ds(row0, ROWS_LAST_TILE)])


def _writeback_my_rows(acc_sh, out, s):
    row0 = s * ROWS_PER_TILE

    @pl.when(s < NUM_SUBCORES - 1)
    def _():
        pltpu.sync_copy(acc_sh.at[pl.ds(row0, ROWS_PER_TILE)],
                        out.at[pl.ds(row0, ROWS_PER_TILE)])

    @pl.when(s == NUM_SUBCORES - 1)
    def _():
        pltpu.sync_copy(acc_sh.at[pl.ds(row0, ROWS_LAST_TILE)],
                        out.at[pl.ds(row0, ROWS_LAST_TILE)])


EDGES_PER_WORKER = N_EDGES // (NUM_SC_CORES * NUM_SUBCORES)
MAIN_BATCH = 80
N_MAIN = 62
TAIL_BATCH = EDGES_PER_WORKER - N_MAIN * MAIN_BATCH


@functools.cache
def _build_segsum():
    @functools.partial(
        pl.kernel,
        out_type=(
            jax.ShapeDtypeStruct((N_NODES, DH), jnp.float32),
            jax.ShapeDtypeStruct((N_NODES, DH), jnp.float32),
        ),
        mesh=_mesh(),
        scratch_types=[
            pltpu.VMEM((N_MAIN, MAIN_BATCH), jnp.int32),
            pltpu.VMEM((N_MAIN, MAIN_BATCH), jnp.int32),
            pltpu.VMEM((1, TAIL_BATCH), jnp.int32),
            pltpu.VMEM((1, TAIL_BATCH), jnp.int32),
            pltpu.VMEM((MAIN_BATCH, DH), jnp.float32),
            pltpu.VMEM((MAIN_BATCH, DH), jnp.float32),
            pltpu.VMEM((MAIN_BATCH, DH), jnp.float32),
            pltpu.VMEM_SHARED((N_NODES, DH), jnp.float32),
            pltpu.SemaphoreType.DMA,
            pltpu.SemaphoreType.DMA,
            pltpu.SemaphoreType.DMA,
            pltpu.SemaphoreType.DMA,
        ],
    )
    def segsum(h, srcm_hbm, dstm_hbm, srct_hbm, dstt_hbm, zer_hbm,
               outa, outb,
               srcm_v, dstm_v, srct_v, dstt_v, r0, r1, r2, acc_sh,
               g0, g1, g2, ss):
        c = lax.axis_index("c")
        s = lax.axis_index("s")
        bufs = (r0, r1, r2)
        gsems = (g0, g1, g2)

        pltpu.sync_copy(srcm_hbm.at[c, s], srcm_v)
        pltpu.sync_copy(dstm_hbm.at[c, s], dstm_v)
        pltpu.sync_copy(srct_hbm.at[c, s], srct_v)
        pltpu.sync_copy(dstt_hbm.at[c, s], dstt_v)

        pltpu.async_copy(h.at[srcm_v.at[0]], r0, g0)
        pltpu.async_copy(h.at[srcm_v.at[1]], r1, g1)
        _zero_my_rows(zer_hbm, acc_sh, s)
        plsc.subcore_barrier()

        def step(j, cur, gcur, prv, tw0):
            pltpu.make_async_copy(h.at[srcm_v.at[j]], cur, gcur).wait()

            @pl.when(j >= 1)
            def _():
                pltpu.make_async_copy(
                    prv, acc_sh.at[dstm_v.at[j - 1]], ss).wait()

            @pl.when(j < N_MAIN - 2)
            def _():
                pltpu.async_copy(h.at[srcm_v.at[j + 2]], prv, tw0)

            pltpu.async_copy(cur, acc_sh.at[dstm_v.at[j]], ss, add=True)

        def body(j, carry):
            @pl.when(j % 3 == 0)
            def _():
                step(j, r0, g0, r2, g2)

            @pl.when(j % 3 == 1)
            def _():
                step(j, r1, g1, r0, g0)

            @pl.when(j % 3 == 2)
            def _():
                step(j, r2, g2, r1, g1)

            return carry

        lax.fori_loop(0, N_MAIN, body, 0)

        pltpu.make_async_copy(
            bufs[(N_MAIN - 1) % 3],
            acc_sh.at[dstm_v.at[N_MAIN - 1]], ss).wait()
        rtail = r0.at[pl.ds(0, TAIL_BATCH)]
        pltpu.async_copy(h.at[srct_v.at[0]], rtail, g0).wait()
        pltpu.sync_copy(rtail, acc_sh.at[dstt_v.at[0]], add=True)
        plsc.subcore_barrier()

        @pl.when(c == 0)
        def _():
            _writeback_my_rows(acc_sh, outa, s)

        @pl.when(c == 1)
        def _():
            _writeback_my_rows(acc_sh, outb, s)

    return segsum


def _segsum(h, srcm, dstm, srct, dstt, zer):
    return _build_segsum()(h, srcm, dstm, srct, dstt, zer)


def _tc_experts_body(x_ref, a0a_ref, a0b_ref, a1a_ref, a1b_ref, noise_ref,
                     wg_ref, wn_ref, thr_ref, w1_ref, w2_ref, u_ref,
                     gates_ref):
    x = x_ref[...]
    z = x + jnp.concatenate([a0a_ref[...] + a0b_ref[...],
                             a1a_ref[...] + a1b_ref[...]], axis=1)

    clean = jnp.dot(x, wg_ref[...], preferred_element_type=jnp.float32)
    t = jnp.dot(x, wn_ref[...], preferred_element_type=jnp.float32)
    std = jnp.log1p(jnp.exp(-jnp.abs(t))) + jnp.maximum(t, 0.0) + 1e-2
    noisy = clean + noise_ref[...] * std
    scores = noisy - thr_ref[...]
    open_mask = (scores > 0.0).astype(jnp.float32)
    m = jnp.max(noisy, axis=1, keepdims=True)
    ex = jnp.exp(noisy - m)
    sm = ex / jnp.sum(ex, axis=1, keepdims=True)
    raw = sm * open_mask
    gates_ref[...] = raw / (jnp.sum(raw, axis=1, keepdims=True) + 1e-9)

    h = jnp.maximum(jnp.dot(z, w1_ref[...], preferred_element_type=jnp.float32),
                    0.0)
    us = [jnp.dot(h[:, e * D_HID:(e + 1) * D_HID], w2_ref[e],
                  preferred_element_type=jnp.float32)
          for e in range(N_EXPERTS)]
    u_ref[...] = jnp.concatenate(us, axis=1)


def _tc_experts(x, a0a, a0b, a1a, a1b, noise, w_gate, w_noise, thr,
                w1cat, w2bd):
    blk = lambda shape: pl.BlockSpec(shape, lambda i: (0, 0))
    return pl.pallas_call(
        _tc_experts_body,
        grid=(NUM_NODE_BLOCKS,),
        in_specs=[
            pl.BlockSpec((NODE_BLOCK, D_IN), lambda i: (i, 0)),
            pl.BlockSpec((NODE_BLOCK, DH), lambda i: (i, 0)),
            pl.BlockSpec((NODE_BLOCK, DH), lambda i: (i, 0)),
            pl.BlockSpec((NODE_BLOCK, DH), lambda i: (i, 0)),
            pl.BlockSpec((NODE_BLOCK, DH), lambda i: (i, 0)),
            pl.BlockSpec((NODE_BLOCK, N_EXPERTS), lambda i: (i, 0)),
            blk((D_IN, N_EXPERTS)),
            blk((D_IN, N_EXPERTS)),
            blk((1, N_EXPERTS)),
            blk((D_IN, N_EXPERTS * D_HID)),
            pl.BlockSpec((N_EXPERTS, D_HID, N_CLASSES), lambda i: (0, 0, 0)),
        ],
        out_specs=[
            pl.BlockSpec((NODE_BLOCK, EC), lambda i: (i, 0)),
            pl.BlockSpec((NODE_BLOCK, N_EXPERTS), lambda i: (i, 0)),
        ],
        out_shape=[
            jax.ShapeDtypeStruct((N_NODES, EC), jnp.float32),
            jax.ShapeDtypeStruct((N_NODES, N_EXPERTS), jnp.float32),
        ],
    )(x, a0a, a0b, a1a, a1b, noise, w_gate, w_noise, thr, w1cat, w2bd)


_REP_FULL = np.kron(np.eye(N_EXPERTS), np.ones((1, N_CLASSES))).astype(np.float32)
_SEL_FULL = np.kron(np.ones((N_EXPERTS, 1)), np.eye(N_CLASSES)).astype(np.float32)


_REP_PAD = np.concatenate(
    [_REP_FULL, np.zeros((N_EXPERTS, 3 * DH - EC), np.float32)], axis=1)
_SEL_PAD = np.concatenate(
    [_SEL_FULL, np.zeros((3 * DH - EC, N_CLASSES), np.float32)], axis=0)


def _tc_combine_body(u_ref, p0a_ref, p0b_ref, p1a_ref, p1b_ref, p2a_ref,
                     p2b_ref, gates_ref, rep_ref, sel_ref, repp_ref,
                     selp_ref, y_ref):
    g = gates_ref[...]
    a2 = jnp.concatenate(
        [p0a_ref[...] + p0b_ref[...], p1a_ref[...] + p1b_ref[...],
         p2a_ref[...] + p2b_ref[...]], axis=1)
    gf = jnp.dot(g, rep_ref[...], preferred_element_type=jnp.float32)
    gp = jnp.dot(g, repp_ref[...], preferred_element_type=jnp.float32)
    y = jnp.dot(u_ref[...] * gf, sel_ref[...],
                preferred_element_type=jnp.float32)
    y += jnp.dot(a2 * gp, selp_ref[...], preferred_element_type=jnp.float32)
    y_ref[...] = y


def _tc_combine(u, p0a, p0b, p1a, p1b, p2a, p2b, gates):
    blk = lambda shape: pl.BlockSpec(shape, lambda i: (0, 0))
    consts = (
        jnp.asarray(_REP_FULL), jnp.asarray(_SEL_FULL),
        jnp.asarray(_REP_PAD), jnp.asarray(_SEL_PAD),
    )
    return pl.pallas_call(
        _tc_combine_body,
        grid=(NUM_NODE_BLOCKS,),
        in_specs=[
            pl.BlockSpec((NODE_BLOCK, EC), lambda i: (i, 0)),
            pl.BlockSpec((NODE_BLOCK, DH), lambda i: (i, 0)),
            pl.BlockSpec((NODE_BLOCK, DH), lambda i: (i, 0)),
            pl.BlockSpec((NODE_BLOCK, DH), lambda i: (i, 0)),
            pl.BlockSpec((NODE_BLOCK, DH), lambda i: (i, 0)),
            pl.BlockSpec((NODE_BLOCK, DH), lambda i: (i, 0)),
            pl.BlockSpec((NODE_BLOCK, DH), lambda i: (i, 0)),
            pl.BlockSpec((NODE_BLOCK, N_EXPERTS), lambda i: (i, 0)),
            blk((N_EXPERTS, EC)), blk((EC, N_CLASSES)),
            blk((N_EXPERTS, 3 * DH)), blk((3 * DH, N_CLASSES)),
        ],
        out_specs=pl.BlockSpec((NODE_BLOCK, N_CLASSES), lambda i: (i, 0)),
        out_shape=jax.ShapeDtypeStruct((N_NODES, N_CLASSES), jnp.float32),
    )(u, p0a, p0b, p1a, p1b, p2a, p2b, gates, *consts)


def kernel(x, edge_index, noise, w_gate, w_noise, gate_threshold, W1, W2):
    ncut = N_MAIN * MAIN_BATCH
    src = edge_index[0].astype(jnp.int32).reshape(
        NUM_SC_CORES, NUM_SUBCORES, EDGES_PER_WORKER)
    dst = edge_index[1].astype(jnp.int32).reshape(
        NUM_SC_CORES, NUM_SUBCORES, EDGES_PER_WORKER)
    srcm = src[:, :, :ncut].reshape(NUM_SC_CORES, NUM_SUBCORES,
                                    N_MAIN, MAIN_BATCH)
    dstm = dst[:, :, :ncut].reshape(NUM_SC_CORES, NUM_SUBCORES,
                                    N_MAIN, MAIN_BATCH)
    srct = src[:, :, ncut:].reshape(NUM_SC_CORES, NUM_SUBCORES,
                                    1, TAIL_BATCH)
    dstt = dst[:, :, ncut:].reshape(NUM_SC_CORES, NUM_SUBCORES,
                                    1, TAIL_BATCH)
    idx = (srcm, dstm, srct, dstt)

    zer = jnp.zeros((N_NODES, DH), jnp.float32)
    x0 = x[:, :DH]
    x1 = x[:, DH:]
    x0a, x0b = _segsum(x0, *idx, zer)
    x1a, x1b = _segsum(x1, *idx, zer)

    w1cat = jnp.transpose(W1, (1, 0, 2)).reshape(D_IN, N_EXPERTS * D_HID)
    thr = gate_threshold.reshape(1, N_EXPERTS)

    u, gates = _tc_experts(x, x0a, x0b, x1a, x1b, noise, w_gate, w_noise,
                           thr, w1cat, W2)

    u0 = u[:, :DH]
    u1 = u[:, DH:2 * DH]
    u2 = jnp.pad(u[:, 2 * DH:], ((0, 0), (0, 3 * DH - EC)))
    p0a, p0b = _segsum(u0, *idx, zer)
    p1a, p1b = _segsum(u1, *idx, zer)
    p2a, p2b = _segsum(u2, *idx, zer)

    return _tc_combine(u, p0a, p0b, p1a, p1b, p2a, p2b, gates)


# speedup vs baseline: 16.1150x; 1.0342x over previous
import functools

import jax
import jax.numpy as jnp
import numpy as np
from jax import lax
from jax.experimental import pallas as pl
from jax.experimental.pallas import tpu as pltpu
from jax.experimental.pallas import tpu_sc as plsc

N_NODES = 10000
N_EDGES = 160000
D_IN = 256
D_HID = 256
N_CLASSES = 40
N_EXPERTS = 8

NUM_SC_CORES = 2
NUM_SUBCORES = 16
EDGES_PER_TILE = N_EDGES // NUM_SUBCORES
EDGE_BATCH = 80
BATCHES_PER_TILE = EDGES_PER_TILE // EDGE_BATCH
ROWS_PER_TILE = 624
ROWS_LAST_TILE = N_NODES - ROWS_PER_TILE * (NUM_SUBCORES - 1)

EC = N_EXPERTS * N_CLASSES

NODE_BLOCK = 1000
NUM_NODE_BLOCKS = N_NODES // NODE_BLOCK


@functools.cache
def _mesh():
    return plsc.VectorSubcoreMesh(
        core_axis_name="c", subcore_axis_name="s",
        num_cores=NUM_SC_CORES, num_subcores=NUM_SUBCORES)


DH = 128


def _zero_my_rows(zer_hbm, acc_sh, s):
    row0 = s * ROWS_PER_TILE

    @pl.when(s < NUM_SUBCORES - 1)
    def _():
        pltpu.sync_copy(zer_hbm.at[pl.ds(row0, ROWS_PER_TILE)],
                        acc_sh.at[pl.ds(row0, ROWS_PER_TILE)])

    @pl.when(s == NUM_SUBCORES - 1)
    def _():
        pltpu.sync_copy(zer_hbm.at[pl.ds(row0, ROWS_LAST_TILE)],
                        acc_sh.at[pl.ds(row0, ROWS_LAST_TILE)])


def _writeback_my_rows(acc_sh, out, s):
    row0 = s * ROWS_PER_TILE

    @pl.when(s < NUM_SUBCORES - 1)
    def _():
        pltpu.sync_copy(acc_sh.at[pl.ds(row0, ROWS_PER_TILE)],
                        out.at[pl.ds(row0, ROWS_PER_TILE)])

    @pl.when(s == NUM_SUBCORES - 1)
    def _():
        pltpu.sync_copy(acc_sh.at[pl.ds(row0, ROWS_LAST_TILE)],
                        out.at[pl.ds(row0, ROWS_LAST_TILE)])


EDGES_PER_WORKER = N_EDGES // (NUM_SC_CORES * NUM_SUBCORES)
MAIN_BATCH = 80
N_MAIN = 62
TAIL_BATCH = EDGES_PER_WORKER - N_MAIN * MAIN_BATCH


@functools.cache
def _build_segsum():
    @functools.partial(
        pl.kernel,
        out_type=(
            jax.ShapeDtypeStruct((N_NODES, DH), jnp.float32),
            jax.ShapeDtypeStruct((N_NODES, DH), jnp.float32),
        ),
        mesh=_mesh(),
        scratch_types=[
            pltpu.VMEM((N_MAIN, MAIN_BATCH), jnp.int32),
            pltpu.VMEM((N_MAIN, MAIN_BATCH), jnp.int32),
            pltpu.VMEM((1, TAIL_BATCH), jnp.int32),
            pltpu.VMEM((1, TAIL_BATCH), jnp.int32),
            pltpu.VMEM((MAIN_BATCH, DH), jnp.float32),
            pltpu.VMEM((MAIN_BATCH, DH), jnp.float32),
            pltpu.VMEM((MAIN_BATCH, DH), jnp.float32),
            pltpu.VMEM_SHARED((N_NODES, DH), jnp.float32),
            pltpu.SemaphoreType.DMA,
            pltpu.SemaphoreType.DMA,
            pltpu.SemaphoreType.DMA,
            pltpu.SemaphoreType.DMA,
        ],
    )
    def segsum(h, srcm_hbm, dstm_hbm, srct_hbm, dstt_hbm, zer_hbm,
               outa, outb,
               srcm_v, dstm_v, srct_v, dstt_v, r0, r1, r2, acc_sh,
               g0, g1, g2, ss):
        c = lax.axis_index("c")
        s = lax.axis_index("s")
        bufs = (r0, r1, r2)
        gsems = (g0, g1, g2)

        pltpu.sync_copy(srcm_hbm.at[c, s], srcm_v)
        pltpu.sync_copy(dstm_hbm.at[c, s], dstm_v)
        pltpu.sync_copy(srct_hbm.at[c, s], srct_v)
        pltpu.sync_copy(dstt_hbm.at[c, s], dstt_v)

        pltpu.async_copy(h.at[srcm_v.at[0]], r0, g0)
        pltpu.async_copy(h.at[srcm_v.at[1]], r1, g1)
        _zero_my_rows(zer_hbm, acc_sh, s)
        plsc.subcore_barrier()

        def step(j, cur, gcur, prv, tw0):
            pltpu.make_async_copy(h.at[srcm_v.at[j]], cur, gcur).wait()

            @pl.when(j >= 1)
            def _():
                pltpu.make_async_copy(
                    prv, acc_sh.at[dstm_v.at[j - 1]], ss).wait()

            @pl.when(j < N_MAIN - 2)
            def _():
                pltpu.async_copy(h.at[srcm_v.at[j + 2]], prv, tw0)

            pltpu.async_copy(cur, acc_sh.at[dstm_v.at[j]], ss, add=True)

        def body(j, carry):
            @pl.when(j % 3 == 0)
            def _():
                step(j, r0, g0, r2, g2)

            @pl.when(j % 3 == 1)
            def _():
                step(j, r1, g1, r0, g0)

            @pl.when(j % 3 == 2)
            def _():
                step(j, r2, g2, r1, g1)

            return carry

        lax.fori_loop(0, N_MAIN, body, 0)

        pltpu.make_async_copy(
            bufs[(N_MAIN - 1) % 3],
            acc_sh.at[dstm_v.at[N_MAIN - 1]], ss).wait()
        rtail = r0.at[pl.ds(0, TAIL_BATCH)]
        pltpu.async_copy(h.at[srct_v.at[0]], rtail, g0).wait()
        pltpu.sync_copy(rtail, acc_sh.at[dstt_v.at[0]], add=True)
        plsc.subcore_barrier()

        @pl.when(c == 0)
        def _():
            _writeback_my_rows(acc_sh, outa, s)

        @pl.when(c == 1)
        def _():
            _writeback_my_rows(acc_sh, outb, s)

    return segsum


def _segsum(h, srcm, dstm, srct, dstt, zer):
    return _build_segsum()(h, srcm, dstm, srct, dstt, zer)


def _tc_experts_body(x_ref, a0a_ref, a0b_ref, a1a_ref, a1b_ref, noise_ref,
                     wg_ref, wn_ref, thr_ref, w1_ref, w2_ref,
                     u0_ref, u1_ref, u2_ref, gates_ref):
    x = x_ref[...]
    z = x + jnp.concatenate([a0a_ref[...] + a0b_ref[...],
                             a1a_ref[...] + a1b_ref[...]], axis=1)

    clean = jnp.dot(x, wg_ref[...], preferred_element_type=jnp.float32)
    t = jnp.dot(x, wn_ref[...], preferred_element_type=jnp.float32)
    std = jnp.log1p(jnp.exp(-jnp.abs(t))) + jnp.maximum(t, 0.0) + 1e-2
    noisy = clean + noise_ref[...] * std
    scores = noisy - thr_ref[...]
    open_mask = (scores > 0.0).astype(jnp.float32)
    m = jnp.max(noisy, axis=1, keepdims=True)
    ex = jnp.exp(noisy - m)
    sm = ex / jnp.sum(ex, axis=1, keepdims=True)
    raw = sm * open_mask
    gates_ref[...] = raw / (jnp.sum(raw, axis=1, keepdims=True) + 1e-9)

    h = jnp.maximum(
        jnp.dot(z.astype(jnp.bfloat16), w1_ref[...],
                preferred_element_type=jnp.float32), 0.0)
    hb = h.astype(jnp.bfloat16)
    us = [jnp.dot(hb[:, e * D_HID:(e + 1) * D_HID], w2_ref[e],
                  preferred_element_type=jnp.float32)
          for e in range(N_EXPERTS)]
    u = jnp.concatenate(
        us + [jnp.zeros((NODE_BLOCK, 3 * DH - EC), jnp.float32)], axis=1)
    u0_ref[...] = u[:, :DH]
    u1_ref[...] = u[:, DH:2 * DH]
    u2_ref[...] = u[:, 2 * DH:]


def _tc_experts(x, a0a, a0b, a1a, a1b, noise, w_gate, w_noise, thr,
                w1cat, w2bd):
    blk = lambda shape: pl.BlockSpec(shape, lambda i: (0, 0))
    return pl.pallas_call(
        _tc_experts_body,
        grid=(NUM_NODE_BLOCKS,),
        in_specs=[
            pl.BlockSpec((NODE_BLOCK, D_IN), lambda i: (i, 0)),
            pl.BlockSpec((NODE_BLOCK, DH), lambda i: (i, 0)),
            pl.BlockSpec((NODE_BLOCK, DH), lambda i: (i, 0)),
            pl.BlockSpec((NODE_BLOCK, DH), lambda i: (i, 0)),
            pl.BlockSpec((NODE_BLOCK, DH), lambda i: (i, 0)),
            pl.BlockSpec((NODE_BLOCK, N_EXPERTS), lambda i: (i, 0)),
            blk((D_IN, N_EXPERTS)),
            blk((D_IN, N_EXPERTS)),
            blk((1, N_EXPERTS)),
            blk((D_IN, N_EXPERTS * D_HID)),
            pl.BlockSpec((N_EXPERTS, D_HID, N_CLASSES), lambda i: (0, 0, 0)),
        ],
        out_specs=[
            pl.BlockSpec((NODE_BLOCK, DH), lambda i: (i, 0)),
            pl.BlockSpec((NODE_BLOCK, DH), lambda i: (i, 0)),
            pl.BlockSpec((NODE_BLOCK, DH), lambda i: (i, 0)),
            pl.BlockSpec((NODE_BLOCK, N_EXPERTS), lambda i: (i, 0)),
        ],
        out_shape=[
            jax.ShapeDtypeStruct((N_NODES, DH), jnp.float32),
            jax.ShapeDtypeStruct((N_NODES, DH), jnp.float32),
            jax.ShapeDtypeStruct((N_NODES, DH), jnp.float32),
            jax.ShapeDtypeStruct((N_NODES, N_EXPERTS), jnp.float32),
        ],
    )(x, a0a, a0b, a1a, a1b, noise, w_gate, w_noise, thr, w1cat, w2bd)


_REP_FULL = np.kron(np.eye(N_EXPERTS), np.ones((1, N_CLASSES))).astype(np.float32)
_SEL_FULL = np.kron(np.ones((N_EXPERTS, 1)), np.eye(N_CLASSES)).astype(np.float32)


_REP_PAD = np.concatenate(
    [_REP_FULL, np.zeros((N_EXPERTS, 3 * DH - EC), np.float32)], axis=1)
_SEL_PAD = np.concatenate(
    [_SEL_FULL, np.zeros((3 * DH - EC, N_CLASSES), np.float32)], axis=0)


def _tc_combine_body(u0_ref, u1_ref, u2_ref, p0a_ref, p0b_ref, p1a_ref,
                     p1b_ref, p2a_ref, p2b_ref, gates_ref, repp_ref,
                     selp_ref, y_ref):
    g = gates_ref[...]
    su = jnp.concatenate(
        [u0_ref[...] + p0a_ref[...] + p0b_ref[...],
         u1_ref[...] + p1a_ref[...] + p1b_ref[...],
         u2_ref[...] + p2a_ref[...] + p2b_ref[...]], axis=1)
    gp = jnp.dot(g, repp_ref[...], preferred_element_type=jnp.float32)
    y_ref[...] = jnp.dot(su * gp, selp_ref[...],
                         preferred_element_type=jnp.float32)


def _tc_combine(u0, u1, u2, p0a, p0b, p1a, p1b, p2a, p2b, gates):
    blk = lambda shape: pl.BlockSpec(shape, lambda i: (0, 0))
    consts = (jnp.asarray(_REP_PAD), jnp.asarray(_SEL_PAD))
    nb = lambda: pl.BlockSpec((NODE_BLOCK, DH), lambda i: (i, 0))
    return pl.pallas_call(
        _tc_combine_body,
        grid=(NUM_NODE_BLOCKS,),
        in_specs=[
            nb(), nb(), nb(), nb(), nb(), nb(), nb(), nb(), nb(),
            pl.BlockSpec((NODE_BLOCK, N_EXPERTS), lambda i: (i, 0)),
            blk((N_EXPERTS, 3 * DH)), blk((3 * DH, N_CLASSES)),
        ],
        out_specs=pl.BlockSpec((NODE_BLOCK, N_CLASSES), lambda i: (i, 0)),
        out_shape=jax.ShapeDtypeStruct((N_NODES, N_CLASSES), jnp.float32),
    )(u0, u1, u2, p0a, p0b, p1a, p1b, p2a, p2b, gates, *consts)


def kernel(x, edge_index, noise, w_gate, w_noise, gate_threshold, W1, W2):
    ncut = N_MAIN * MAIN_BATCH
    src = edge_index[0].astype(jnp.int32).reshape(
        NUM_SC_CORES, NUM_SUBCORES, EDGES_PER_WORKER)
    dst = edge_index[1].astype(jnp.int32).reshape(
        NUM_SC_CORES, NUM_SUBCORES, EDGES_PER_WORKER)
    srcm = src[:, :, :ncut].reshape(NUM_SC_CORES, NUM_SUBCORES,
                                    N_MAIN, MAIN_BATCH)
    dstm = dst[:, :, :ncut].reshape(NUM_SC_CORES, NUM_SUBCORES,
                                    N_MAIN, MAIN_BATCH)
    srct = src[:, :, ncut:].reshape(NUM_SC_CORES, NUM_SUBCORES,
                                    1, TAIL_BATCH)
    dstt = dst[:, :, ncut:].reshape(NUM_SC_CORES, NUM_SUBCORES,
                                    1, TAIL_BATCH)
    idx = (srcm, dstm, srct, dstt)

    zer = jnp.zeros((N_NODES, DH), jnp.float32)
    x0 = x[:, :DH]
    x1 = x[:, DH:]
    x0a, x0b = _segsum(x0, *idx, zer)
    x1a, x1b = _segsum(x1, *idx, zer)

    w1cat = jnp.transpose(W1, (1, 0, 2)).reshape(
        D_IN, N_EXPERTS * D_HID).astype(jnp.bfloat16)
    w2b = W2.astype(jnp.bfloat16)
    thr = gate_threshold.reshape(1, N_EXPERTS)

    u0, u1, u2, gates = _tc_experts(x, x0a, x0b, x1a, x1b, noise, w_gate,
                                    w_noise, thr, w1cat, w2b)

    p0a, p0b = _segsum(u0, *idx, zer)
    p1a, p1b = _segsum(u1, *idx, zer)
    p2a, p2b = _segsum(u2, *idx, zer)

    return _tc_combine(u0, u1, u2, p0a, p0b, p1a, p1b, p2a, p2b, gates)
